# speedup vs baseline: 1.0764x; 1.0764x over previous
import functools

import jax
import jax.numpy as jnp
from jax import lax
from jax.experimental import pallas as pl
from jax.experimental.pallas import tpu as pltpu

_LANE = 128


def _round_up(x, m):
    return (x + m - 1) // m * m


def _pick_row_tile(h, target=8):
    t = min(h, target)
    while h % t:
        t -= 1
    return t


def _conv_stats_kernel(xpad_ref, w_ref, conv_ref, stats_ref, *,
                       TH, Wo, Cinp, Coutp, KH, KW, dilation):
    h = pl.program_id(1)
    row0 = pl.multiple_of(h * TH, TH)

    patches = []
    for kh in range(KH):
        r0 = row0 + kh * dilation
        for kw in range(KW):
            c0 = kw * dilation
            p = xpad_ref[pl.ds(r0, TH), pl.ds(c0, Wo), :]
            patches.append(p.reshape(TH * Wo, Cinp))
    lhs = jnp.concatenate(patches, axis=-1)

    acc = jnp.dot(lhs, w_ref[...], preferred_element_type=jnp.float32)

    conv_ref[...] = acc.reshape(TH, Wo, Coutp).astype(conv_ref.dtype)

    s = jnp.sum(acc, axis=0, keepdims=True)
    ss = jnp.sum(acc * acc, axis=0, keepdims=True)
    stats_ref[...] = jnp.concatenate([s, ss], axis=0)


def _bn_relu_kernel(conv_ref, scale_ref, shift_ref, out_ref):
    y = conv_ref[...].astype(jnp.float32) * scale_ref[...] + shift_ref[...]
    out_ref[...] = jnp.maximum(y, 0.0).astype(out_ref.dtype)


def kernel(x_nchw, weight_oihw, gamma, beta):
    padding, dilation, eps = 2, 2, 1e-5
    N, Cin, H, W = x_nchw.shape
    Cout, _, KH, KW = weight_oihw.shape

    Ho = H + 2 * padding - dilation * (KH - 1)
    Wo = W + 2 * padding - dilation * (KW - 1)
    Hp, Wp = H + 2 * padding, W + 2 * padding

    Cinp = _round_up(Cin, _LANE)
    Coutp = _round_up(Cout, _LANE)

    x_nhwc = jnp.transpose(x_nchw, (0, 2, 3, 1))
    xpad = jnp.pad(x_nhwc, ((0, 0), (padding, padding), (padding, padding),
                            (0, Cinp - Cin))).astype(jnp.bfloat16)

    w = jnp.transpose(weight_oihw, (2, 3, 1, 0))
    w = jnp.pad(w, ((0, 0), (0, 0), (0, Cinp - Cin), (0, Coutp - Cout)))
    w2d = w.reshape(KH * KW * Cinp, Coutp).astype(jnp.bfloat16)

    TH = _pick_row_tile(Ho, target=8)
    Hg = Ho // TH

    conv_kernel = functools.partial(
        _conv_stats_kernel, TH=TH, Wo=Wo, Cinp=Cinp, Coutp=Coutp,
        KH=KH, KW=KW, dilation=dilation)

    conv, stats = pl.pallas_call(
        conv_kernel,
        out_shape=(jax.ShapeDtypeStruct((N, Ho, Wo, Coutp), jnp.bfloat16),
                   jax.ShapeDtypeStruct((N, Hg, 2, Coutp), jnp.float32)),
        grid=(N, Hg),
        in_specs=[
            pl.BlockSpec((None, Hp, Wp, Cinp), lambda n, h: (n, 0, 0, 0)),
            pl.BlockSpec((KH * KW * Cinp, Coutp), lambda n, h: (0, 0)),
        ],
        out_specs=(
            pl.BlockSpec((None, TH, Wo, Coutp), lambda n, h: (n, h, 0, 0)),
            pl.BlockSpec((None, None, 2, Coutp), lambda n, h: (n, h, 0, 0)),
        ),
        compiler_params=pltpu.CompilerParams(
            dimension_semantics=("parallel", "parallel"),
            vmem_limit_bytes=32 * 1024 * 1024),
    )(xpad, w2d)

    cnt = float(N * Ho * Wo)
    total = jnp.sum(stats, axis=(0, 1))
    mean = total[0] / cnt
    var = jnp.maximum(total[1] / cnt - mean * mean, 0.0)
    inv = lax.rsqrt(var + eps)
    gamma_p = jnp.pad(gamma.astype(jnp.float32), (0, Coutp - Cout))
    beta_p = jnp.pad(beta.astype(jnp.float32), (0, Coutp - Cout))
    scale = (gamma_p * inv).reshape(1, Coutp)
    shift = (beta_p - mean * gamma_p * inv).reshape(1, Coutp)

    out_nhwc = pl.pallas_call(
        _bn_relu_kernel,
        out_shape=jax.ShapeDtypeStruct((N, Ho, Wo, Coutp), x_nchw.dtype),
        grid=(N, Hg),
        in_specs=[
            pl.BlockSpec((None, TH, Wo, Coutp), lambda n, h: (n, h, 0, 0)),
            pl.BlockSpec((1, Coutp), lambda n, h: (0, 0)),
            pl.BlockSpec((1, Coutp), lambda n, h: (0, 0)),
        ],
        out_specs=pl.BlockSpec((None, TH, Wo, Coutp), lambda n, h: (n, h, 0, 0)),
        compiler_params=pltpu.CompilerParams(
            dimension_semantics=("parallel", "parallel")),
    )(conv, scale, shift)

    return jnp.transpose(out_nhwc[..., :Cout], (0, 3, 1, 2))
